# Initial kernel scaffold; baseline (speedup 1.0000x reference)
import jax
import jax.numpy as jnp
from jax.experimental import pallas as pl


def kernel(x, table, W1, b1, W2, b2):
    raise NotImplementedError("write your pallas kernel here")


# speedup vs baseline: 19.5571x; 19.5571x over previous
import functools

import jax
import jax.numpy as jnp
from jax import lax
from jax.experimental import pallas as pl
from jax.experimental.pallas import tpu as pltpu
from jax.experimental.pallas import tpu_sc as plsc

B = 16384
L = 200
V = 20000
D = 50
DP = 64
NC, NS = 2, 16
NW = NC * NS
RPW = B // NW
G = 100
CH = 64
NCH = RPW // CH


def _mesh():
    return plsc.VectorSubcoreMesh(
        core_axis_name="c", subcore_axis_name="s", num_cores=NC, num_subcores=NS
    )


@functools.partial(
    pl.kernel,
    out_type=jax.ShapeDtypeStruct((B,), jnp.float32),
    mesh=_mesh(),
    scratch_types=[
        pltpu.VMEM((2 * CH, G), jnp.int32),
        pltpu.VMEM((2, G, DP), jnp.float32),
        pltpu.VMEM((RPW,), jnp.float32),
        pltpu.VMEM((2, DP), jnp.float32),
        pltpu.VMEM((16,), jnp.float32),
        pltpu.SemaphoreType.DMA,
        pltpu.SemaphoreType.DMA,
    ],
    compiler_params=pltpu.CompilerParams(
        needs_layout_passes=False, use_tc_tiling_on_sc=False
    ),
)
def _sc_embed_mlp(x2_hbm, table_hbm, w1_hbm, par_hbm, out_hbm,
                  idx_v, db, out_v, w1_v, par_v, sem0, sem1):
    wid = lax.axis_index("s") * NC + lax.axis_index("c")
    sems = (sem0, sem1)

    pltpu.sync_copy(w1_hbm, w1_v)
    pltpu.sync_copy(par_hbm, par_v)

    w1c = [[w1_v[j, pl.ds(16 * k, 16)] for k in range(4)] for j in range(2)]
    pv = par_v[pl.ds(0, 16)]
    b1_0 = pv[0]
    b1_1 = pv[1]
    w2_0 = pv[2]
    w2_1 = pv[3]
    b2_0 = pv[4]
    lane_iota = lax.iota(jnp.int32, 16)

    def issue(g, b):
        pltpu.async_copy(table_hbm.at[idx_v.at[g]], db.at[b], sems[b])

    def wait(g, b):
        pltpu.make_async_copy(table_hbm.at[idx_v.at[g]], db.at[b], sems[b]).wait()

    def accumulate(b, acc):
        def body(jc, a):
            a = list(a)
            for u in range(4):
                j = jc * 4 + u
                for k in range(4):
                    a[k] = a[k] + db[b, j, pl.ds(16 * k, 16)]
            return tuple(a)
        return lax.fori_loop(0, G // 4, body, acc)

    def finish_row(r, acc, vec):
        m = [jnp.maximum(a, 0.0) for a in acc]
        t0 = m[0] * w1c[0][0] + m[1] * w1c[0][1] + m[2] * w1c[0][2] + m[3] * w1c[0][3]
        t1 = m[0] * w1c[1][0] + m[1] * w1c[1][1] + m[2] * w1c[1][2] + m[3] * w1c[1][3]
        e0 = jnp.sum(t0) + b1_0
        e1 = jnp.sum(t1) + b1_1
        l0 = jnp.maximum(e0, 0.0) + 0.01 * jnp.minimum(e0, 0.0)
        l1 = jnp.maximum(e1, 0.0) + 0.01 * jnp.minimum(e1, 0.0)
        res = l0 * w2_0 + l1 * w2_1 + b2_0
        lane = r & 15
        vec = jnp.where(lane_iota == lane, res, vec)

        @pl.when(lane == 15)
        def _():
            out_v[pl.ds(r - 15, 16)] = vec

        return vec

    zero = jnp.zeros((16,), jnp.float32)

    def chunk(c, vec):
        pltpu.sync_copy(x2_hbm.at[pl.ds(wid * 2 * RPW + c * 2 * CH, 2 * CH)], idx_v)
        issue(0, 0)
        issue(1, 1)

        def row(i, carry):
            vec, acc = carry
            for h in range(2):
                g = 2 * i + h
                wait(g, h)
                acc = accumulate(h, acc)

                @pl.when(g + 2 < 2 * CH)
                def _():
                    issue(g + 2, h)

                if h == 1:
                    vec = finish_row(c * CH + i, acc, vec)
                    acc = (zero, zero, zero, zero)
            return vec, acc

        vec, _ = lax.fori_loop(0, CH, row, (vec, (zero, zero, zero, zero)))
        return vec

    lax.fori_loop(0, NCH, chunk, zero)

    pltpu.sync_copy(out_v, out_hbm.at[pl.ds(wid * RPW, RPW)])


def kernel(x, table, W1, b1, W2, b2):
    x2 = x.astype(jnp.int32).reshape(2 * B, G)
    tpad = jnp.pad(table, ((0, 0), (0, DP - D)))
    w1s = (jnp.pad(W1, ((0, DP - D), (0, 0))).T * (1.0 / L)).astype(jnp.float32)
    par = jnp.concatenate(
        [b1, W2[:, 0], b2, jnp.zeros((11,), jnp.float32)]
    ).astype(jnp.float32)
    return _sc_embed_mlp(x2, tpad, w1s, par)
